# Initial kernel scaffold; baseline (speedup 1.0000x reference)
import jax
import jax.numpy as jnp
from jax.experimental import pallas as pl


def kernel(x, edge_index, W, att_src, att_dst, bias):
    raise NotImplementedError("write your pallas kernel here")


# speedup vs baseline: 1.0098x; 1.0098x over previous
import functools
import jax
import jax.numpy as jnp
import numpy as np
from jax.experimental import pallas as pl
from jax.experimental.pallas import tpu as pltpu

HEADS = 4
CH = 32


def _mm_kernel(x_ref, w_ref, out_ref):
    out_ref[...] = jnp.dot(x_ref[...], w_ref[...],
                           preferred_element_type=jnp.float32)


def _tc_matmul(x_flat, wcat):
    n, d = x_flat.shape
    dout = wcat.shape[1]
    bm = 2048
    grid = (n // bm,)
    return pl.pallas_call(
        _mm_kernel,
        grid=grid,
        in_specs=[
            pl.BlockSpec((bm, d), lambda i: (i, 0)),
            pl.BlockSpec((d, dout), lambda i: (0, 0)),
        ],
        out_specs=pl.BlockSpec((bm, dout), lambda i: (i, 0)),
        out_shape=jax.ShapeDtypeStruct((n, dout), jnp.float32),
    )(x_flat, wcat)


def kernel(x, edge_index, W, att_src, att_dst, bias):
    b, t, j, d = x.shape
    n = b * t * j
    x_flat = x.reshape(n, d)
    e = edge_index.shape[1]

    asrc_mat = jax.scipy.linalg.block_diag(
        *[att_src[0, h, :, None] for h in range(HEADS)])
    adst_mat = jax.scipy.linalg.block_diag(
        *[att_dst[0, h, :, None] for h in range(HEADS)])
    wcat = jnp.concatenate([W, W @ asrc_mat, W @ adst_mat], axis=1)

    feat = _tc_matmul(x_flat, wcat)
    xh = feat[:, :d].reshape(n, HEADS, CH)
    a_src = feat[:, d:d + HEADS]
    a_dst = feat[:, d + HEADS:]

    offsets = jnp.repeat(jnp.arange(b * t, dtype=edge_index.dtype), e) * j
    rep = jnp.tile(edge_index, (1, b * t)) + offsets[None, :]
    loop = jnp.arange(n, dtype=rep.dtype)
    src = jnp.concatenate([rep[0], loop])
    dst = jnp.concatenate([rep[1], loop])
    alpha = jax.nn.leaky_relu(a_src[src] + a_dst[dst], 0.2)
    amax = jax.ops.segment_max(alpha, dst, num_segments=n)
    amax = jnp.where(jnp.isfinite(amax), amax, 0.0)
    ex = jnp.exp(alpha - amax[dst])
    denom = jax.ops.segment_sum(ex, dst, num_segments=n)
    coef = ex / (denom[dst] + 1e-16)
    out = jax.ops.segment_sum(xh[src] * coef[:, :, None], dst, num_segments=n)
    out = out.reshape(n, HEADS * CH) + bias
    return out.reshape(b, t, j, d)


# speedup vs baseline: 64.9643x; 64.3308x over previous
import functools
import jax
import jax.numpy as jnp
import numpy as np
from jax import lax
from jax.experimental import pallas as pl
from jax.experimental.pallas import tpu as pltpu
from jax.experimental.pallas import tpu_sc as plsc

NC = 2
NS = 16
NW = NC * NS
LANES = 16


def _mm_kernel(x_ref, w_ref, out_ref):
    out_ref[...] = jnp.dot(x_ref[...], w_ref[...],
                           preferred_element_type=jnp.float32)


def _tc_matmul(x_flat, wcat):
    n, d = x_flat.shape
    dout = wcat.shape[1]
    bm = 2048
    return pl.pallas_call(
        _mm_kernel,
        grid=(n // bm,),
        in_specs=[
            pl.BlockSpec((bm, d), lambda i: (i, 0)),
            pl.BlockSpec((d, dout), lambda i: (0, 0)),
        ],
        out_specs=pl.BlockSpec((bm, dout), lambda i: (i, 0)),
        out_shape=jax.ShapeDtypeStruct((n, dout), jnp.float32),
    )(x_flat, wcat)


def _make_sc_kernel(n, j, d, heads, F, EPAD, gpw):
    ngrp = EPAD // LANES
    jF = j * F
    jd = j * d
    ch = d // heads

    mesh = plsc.VectorSubcoreMesh(core_axis_name="c", subcore_axis_name="s")

    @functools.partial(
        pl.kernel, mesh=mesh,
        compiler_params=pltpu.CompilerParams(needs_layout_passes=False),
        out_type=jax.ShapeDtypeStruct((n * d,), jnp.float32),
        scratch_types=[
            pltpu.VMEM((jF,), jnp.float32),
            pltpu.VMEM((jF,), jnp.float32),
            pltpu.VMEM((jd,), jnp.float32),
            pltpu.VMEM((4 * EPAD,), jnp.int32),
            pltpu.VMEM((EPAD,), jnp.float32),
            pltpu.VMEM((d,), jnp.float32),
        ],
    )
    def sc_k(feat_hbm, eint_hbm, maskf_hbm, bias_hbm, out_hbm,
             featv, accf, outv, eintv, maskv, biasv):
        wid = lax.axis_index("s") * NC + lax.axis_index("c")
        pltpu.sync_copy(eint_hbm, eintv)
        pltpu.sync_copy(maskf_hbm, maskv)
        pltpu.sync_copy(bias_hbm, biasv)

        zv = jnp.zeros((LANES,), jnp.float32)
        for k in range(jF // LANES):
            accf[pl.ds(LANES * k, LANES)] = zv

        def body(i, carry):
            gid = wid * gpw + i
            pltpu.sync_copy(feat_hbm.at[pl.ds(gid * jF, jF)], featv)

            smax = [jnp.full((LANES,), -1e30, jnp.float32)
                    for _ in range(heads)]
            for g in range(ngrp):
                s144v = eintv[pl.ds(g * LANES, LANES)]
                d144v = eintv[pl.ds(EPAD + g * LANES, LANES)]
                mv = maskv[pl.ds(g * LANES, LANES)]
                nm = (mv - 1.0) * 1e30
                for h in range(heads):
                    av = plsc.load_gather(featv, [s144v + (d + h)])
                    bv = plsc.load_gather(featv, [d144v + (d + heads + h)])
                    a = av + bv
                    a = jnp.where(a >= 0.0, a, a * 0.2)
                    am = a * mv + nm
                    smax[h] = jnp.maximum(smax[h], am)
            sm = [jnp.max(smax[h]) for h in range(heads)]

            iotav = lax.iota(jnp.int32, LANES)
            m4 = iotav < heads

            def edge_body(e_, c2):
                ev = jnp.full((LANES,), 0, jnp.int32) + e_
                ssplat = plsc.load_gather(eintv, [ev])
                dsplat = plsc.load_gather(eintv, [ev + EPAD])
                mk = plsc.load_gather(maskv, [ev])
                exs = []
                for h in range(heads):
                    av = plsc.load_gather(featv, [ssplat + (d + h)])
                    bv = plsc.load_gather(featv, [dsplat + (d + heads + h)])
                    a = av + bv
                    a = jnp.where(a >= 0.0, a, a * 0.2)
                    exs.append(jnp.exp(a - sm[h]) * mk)
                ex4 = exs[0]
                for h in range(1, heads):
                    ex4 = jnp.where(iotav == h, exs[h], ex4)
                plsc.addupdate_scatter(accf, [dsplat + (iotav + d)],
                                       ex4, mask=m4)
                for q in range(d // LANES):
                    cq = iotav + q * LANES
                    xv = plsc.load_gather(featv, [ssplat + cq])
                    plsc.addupdate_scatter(accf, [dsplat + cq],
                                           xv * exs[(q * LANES) // ch])
                return c2

            lax.fori_loop(0, EPAD, edge_body, jnp.int32(0))

            def node_body(nd, c2):
                rec = []
                for h in range(heads):
                    dh = plsc.load_gather(
                        accf,
                        [jnp.full((LANES,), d + h, jnp.int32) + nd * F])
                    rec.append(1.0 / (dh + 1e-16))
                for q in range(d // LANES):
                    acc = accf[pl.ds(nd * F + q * LANES, LANES)]
                    outv[pl.ds(nd * d + q * LANES, LANES)] = (
                        acc * rec[(q * LANES) // ch]
                        + biasv[pl.ds(q * LANES, LANES)])
                    accf[pl.ds(nd * F + q * LANES, LANES)] = zv
                accf[pl.ds(nd * F + d, LANES)] = zv
                return c2

            lax.fori_loop(0, j, node_body, jnp.int32(0))

            pltpu.sync_copy(outv, out_hbm.at[pl.ds(gid * jd, jd)])
            return carry

        lax.fori_loop(0, gpw, body, jnp.int32(0))

    return sc_k


def kernel(x, edge_index, W, att_src, att_dst, bias):
    b, t, j, d = x.shape
    n = b * t * j
    ng = b * t
    e = edge_index.shape[1]
    heads = att_src.shape[1]
    F = 144

    x_flat = x.reshape(n, d)
    asrc_mat = jax.scipy.linalg.block_diag(
        *[att_src[0, h, :, None] for h in range(heads)])
    adst_mat = jax.scipy.linalg.block_diag(
        *[att_dst[0, h, :, None] for h in range(heads)])
    wcat = jnp.concatenate(
        [W, W @ asrc_mat, W @ adst_mat,
         jnp.zeros((d, F - d - 2 * heads), jnp.float32)], axis=1)
    feat = _tc_matmul(x_flat, wcat)

    ne = e + j
    EPAD = ((ne + LANES - 1) // LANES) * LANES
    src = jnp.concatenate([edge_index[0],
                           jnp.arange(j, dtype=jnp.int32)])
    dst = jnp.concatenate([edge_index[1],
                           jnp.arange(j, dtype=jnp.int32)])
    pad = jnp.zeros((EPAD - ne,), jnp.int32)
    src = jnp.concatenate([src, pad])
    dst = jnp.concatenate([dst, pad])
    eint = jnp.concatenate([src * F, dst * F, dst * d, dst])
    maskf = (jnp.arange(EPAD) < ne).astype(jnp.float32)

    gpw = ng // NW
    sc_k = _make_sc_kernel(n, j, d, heads, F, EPAD, gpw)
    out_f = sc_k(feat.reshape(-1), eint, maskf, bias)
    return out_f.reshape(b, t, j, d)


# speedup vs baseline: 103.8325x; 1.5983x over previous
import functools
import jax
import jax.numpy as jnp
import numpy as np
from jax import lax
from jax.experimental import pallas as pl
from jax.experimental.pallas import tpu as pltpu
from jax.experimental.pallas import tpu_sc as plsc

NC = 2
NS = 16
NW = NC * NS
LANES = 16


def _mm_kernel(x_ref, w_ref, out_ref):
    out_ref[...] = jnp.dot(x_ref[...], w_ref[...],
                           preferred_element_type=jnp.float32)


def _tc_matmul(x_flat, wcat):
    n, d = x_flat.shape
    dout = wcat.shape[1]
    bm = 2048
    return pl.pallas_call(
        _mm_kernel,
        grid=(n // bm,),
        in_specs=[
            pl.BlockSpec((bm, d), lambda i: (i, 0)),
            pl.BlockSpec((d, dout), lambda i: (0, 0)),
        ],
        out_specs=pl.BlockSpec((bm, dout), lambda i: (i, 0)),
        out_shape=jax.ShapeDtypeStruct((n, dout), jnp.float32),
    )(x_flat, wcat)


def _make_sc_kernel(n, j, d, heads, F, EPAD, gpw):
    ngrp = EPAD // LANES
    jF = j * F
    jd = j * d
    ch = d // heads

    mesh = plsc.VectorSubcoreMesh(core_axis_name="c", subcore_axis_name="s")

    @functools.partial(
        pl.kernel, mesh=mesh,
        compiler_params=pltpu.CompilerParams(needs_layout_passes=False),
        out_type=jax.ShapeDtypeStruct((n * d,), jnp.float32),
        scratch_types=[
            pltpu.VMEM((jF,), jnp.float32),
            pltpu.VMEM((jF,), jnp.float32),
            pltpu.VMEM((jd,), jnp.float32),
            pltpu.VMEM((4 * EPAD,), jnp.int32),
            pltpu.VMEM((EPAD,), jnp.float32),
            pltpu.VMEM((d,), jnp.float32),
        ],
    )
    def sc_k(feat_hbm, eint_hbm, maskf_hbm, bias_hbm, out_hbm,
             featv, accf, outv, eintv, maskv, biasv):
        wid = lax.axis_index("s") * NC + lax.axis_index("c")
        pltpu.sync_copy(eint_hbm, eintv)
        pltpu.sync_copy(maskf_hbm, maskv)
        pltpu.sync_copy(bias_hbm, biasv)

        zv = jnp.zeros((LANES,), jnp.float32)
        for k in range(jF // LANES):
            accf[pl.ds(LANES * k, LANES)] = zv

        def body(i, carry):
            gid = wid * gpw + i
            pltpu.sync_copy(feat_hbm.at[pl.ds(gid * jF, jF)], featv)

            smax = [jnp.full((LANES,), -1e30, jnp.float32)
                    for _ in range(heads)]
            for g in range(ngrp):
                s144v = eintv[pl.ds(g * LANES, LANES)]
                d144v = eintv[pl.ds(EPAD + g * LANES, LANES)]
                mv = maskv[pl.ds(g * LANES, LANES)]
                nm = (mv - 1.0) * 1e30
                for h in range(heads):
                    av = plsc.load_gather(featv, [s144v + (d + h)])
                    bv = plsc.load_gather(featv, [d144v + (d + heads + h)])
                    a = av + bv
                    a = jnp.where(a >= 0.0, a, a * 0.2)
                    am = a * mv + nm
                    smax[h] = jnp.maximum(smax[h], am)
            sm = [jnp.max(smax[h]) for h in range(heads)]

            iotav = lax.iota(jnp.int32, LANES)
            m4 = iotav < heads

            @plsc.parallel_loop(0, EPAD, unroll=4)
            def edge_body(e_):
                ev = jnp.full((LANES,), 0, jnp.int32) + e_
                ssplat = plsc.load_gather(eintv, [ev])
                dsplat = plsc.load_gather(eintv, [ev + EPAD])
                mk = plsc.load_gather(maskv, [ev])
                exs = []
                for h in range(heads):
                    av = plsc.load_gather(featv, [ssplat + (d + h)])
                    bv = plsc.load_gather(featv, [dsplat + (d + heads + h)])
                    a = av + bv
                    a = jnp.where(a >= 0.0, a, a * 0.2)
                    exs.append(jnp.exp(a - sm[h]) * mk)
                ex4 = exs[0]
                for h in range(1, heads):
                    ex4 = jnp.where(iotav == h, exs[h], ex4)
                plsc.addupdate_scatter(accf, [dsplat + (iotav + d)],
                                       ex4, mask=m4)
                for q in range(d // LANES):
                    cq = iotav + q * LANES
                    xv = plsc.load_gather(featv, [ssplat + cq])
                    plsc.addupdate_scatter(accf, [dsplat + cq],
                                           xv * exs[(q * LANES) // ch])

            def node_body(nd, c2):
                rec = []
                for h in range(heads):
                    dh = plsc.load_gather(
                        accf,
                        [jnp.full((LANES,), d + h, jnp.int32) + nd * F])
                    rec.append(1.0 / (dh + 1e-16))
                for q in range(d // LANES):
                    acc = accf[pl.ds(nd * F + q * LANES, LANES)]
                    outv[pl.ds(nd * d + q * LANES, LANES)] = (
                        acc * rec[(q * LANES) // ch]
                        + biasv[pl.ds(q * LANES, LANES)])
                    accf[pl.ds(nd * F + q * LANES, LANES)] = zv
                accf[pl.ds(nd * F + d, LANES)] = zv
                return c2

            lax.fori_loop(0, j, node_body, jnp.int32(0))

            pltpu.sync_copy(outv, out_hbm.at[pl.ds(gid * jd, jd)])
            return carry

        lax.fori_loop(0, gpw, body, jnp.int32(0))

    return sc_k


def kernel(x, edge_index, W, att_src, att_dst, bias):
    b, t, j, d = x.shape
    n = b * t * j
    ng = b * t
    e = edge_index.shape[1]
    heads = att_src.shape[1]
    F = 144

    x_flat = x.reshape(n, d)
    asrc_mat = jax.scipy.linalg.block_diag(
        *[att_src[0, h, :, None] for h in range(heads)])
    adst_mat = jax.scipy.linalg.block_diag(
        *[att_dst[0, h, :, None] for h in range(heads)])
    wcat = jnp.concatenate(
        [W, W @ asrc_mat, W @ adst_mat,
         jnp.zeros((d, F - d - 2 * heads), jnp.float32)], axis=1)
    feat = _tc_matmul(x_flat, wcat)

    ne = e + j
    EPAD = ((ne + LANES - 1) // LANES) * LANES
    src = jnp.concatenate([edge_index[0],
                           jnp.arange(j, dtype=jnp.int32)])
    dst = jnp.concatenate([edge_index[1],
                           jnp.arange(j, dtype=jnp.int32)])
    pad = jnp.zeros((EPAD - ne,), jnp.int32)
    src = jnp.concatenate([src, pad])
    dst = jnp.concatenate([dst, pad])
    eint = jnp.concatenate([src * F, dst * F, dst * d, dst])
    maskf = (jnp.arange(EPAD) < ne).astype(jnp.float32)

    gpw = ng // NW
    sc_k = _make_sc_kernel(n, j, d, heads, F, EPAD, gpw)
    out_f = sc_k(feat.reshape(-1), eint, maskf, bias)
    return out_f.reshape(b, t, j, d)


# speedup vs baseline: 135.1923x; 1.3020x over previous
import functools
import jax
import jax.numpy as jnp
import numpy as np
from jax import lax
from jax.experimental import pallas as pl
from jax.experimental.pallas import tpu as pltpu
from jax.experimental.pallas import tpu_sc as plsc

NC = 2
NS = 16
NW = NC * NS
LANES = 16


def _mm_kernel(x_ref, w_ref, out_ref):
    out_ref[...] = jnp.dot(x_ref[...], w_ref[...],
                           preferred_element_type=jnp.float32)


def _tc_matmul(x_flat, wcat):
    n, d = x_flat.shape
    dout = wcat.shape[1]
    bm = 2048
    return pl.pallas_call(
        _mm_kernel,
        grid=(n // bm,),
        in_specs=[
            pl.BlockSpec((bm, d), lambda i: (i, 0)),
            pl.BlockSpec((d, dout), lambda i: (0, 0)),
        ],
        out_specs=pl.BlockSpec((bm, dout), lambda i: (i, 0)),
        out_shape=jax.ShapeDtypeStruct((n, dout), jnp.float32),
    )(x_flat, wcat)


def _make_sc_kernel(n, j, d, heads, F, EPAD, ne, gpw):
    ngrp = EPAD // LANES
    jF = j * F
    jd = j * d
    ch = d // heads

    mesh = plsc.VectorSubcoreMesh(core_axis_name="c", subcore_axis_name="s")

    @functools.partial(
        pl.kernel, mesh=mesh,
        compiler_params=pltpu.CompilerParams(needs_layout_passes=False),
        out_type=jax.ShapeDtypeStruct((n * d,), jnp.float32),
        scratch_types=[
            pltpu.VMEM((jF,), jnp.float32),
            pltpu.VMEM((jF,), jnp.float32),
            pltpu.VMEM((jd,), jnp.float32),
            pltpu.VMEM((4 * EPAD,), jnp.int32),
            pltpu.VMEM((EPAD,), jnp.float32),
            pltpu.VMEM((d,), jnp.float32),
        ],
    )
    def sc_k(feat_hbm, eint_hbm, maskf_hbm, bias_hbm, out_hbm,
             featv, accf, outv, eintv, maskv, biasv):
        wid = lax.axis_index("s") * NC + lax.axis_index("c")
        pltpu.sync_copy(eint_hbm, eintv)
        pltpu.sync_copy(maskf_hbm, maskv)
        pltpu.sync_copy(bias_hbm, biasv)

        zv = jnp.zeros((LANES,), jnp.float32)
        for k in range(jF // LANES):
            accf[pl.ds(LANES * k, LANES)] = zv

        def body(i, carry):
            gid = wid * gpw + i
            pltpu.sync_copy(feat_hbm.at[pl.ds(gid * jF, jF)], featv)

            smax = [jnp.full((LANES,), -1e30, jnp.float32)
                    for _ in range(heads)]
            for g in range(ngrp):
                s144v = eintv[pl.ds(g * LANES, LANES)]
                d144v = eintv[pl.ds(EPAD + g * LANES, LANES)]
                mv = maskv[pl.ds(g * LANES, LANES)]
                nm = (mv - 1.0) * 1e30
                for h in range(heads):
                    av = plsc.load_gather(featv, [s144v + (d + h)])
                    bv = plsc.load_gather(featv, [d144v + (d + heads + h)])
                    a = av + bv
                    a = jnp.where(a >= 0.0, a, a * 0.2)
                    am = a * mv + nm
                    smax[h] = jnp.maximum(smax[h], am)
            sm = [jnp.max(smax[h]) for h in range(heads)]

            iotav = lax.iota(jnp.int32, LANES)
            m4 = iotav < heads
            hoff4 = iotav & (heads - 1)
            smv = jnp.full((LANES,), 0.0, jnp.float32) + sm[0]
            for h in range(1, heads):
                smv = jnp.where(hoff4 == h, sm[h], smv)

            @plsc.parallel_loop(0, EPAD, unroll=4)
            def edge_body(e_):
                ev = jnp.full((LANES,), 0, jnp.int32) + e_
                ssplat = plsc.load_gather(eintv, [ev])
                dsplat = plsc.load_gather(eintv, [ev + EPAD])
                mk = jnp.where(e_ < ne, 1.0, 0.0)
                av = plsc.load_gather(featv, [ssplat + (d + hoff4)])
                bv = plsc.load_gather(featv, [dsplat + ((d + heads) + hoff4)])
                a = av + bv
                a = jnp.where(a >= 0.0, a, a * 0.2)
                ex4 = jnp.exp(a - smv) * mk
                plsc.addupdate_scatter(accf, [dsplat + (iotav + d)],
                                       ex4, mask=m4)
                exs = [jnp.take_along_axis(
                    ex4, jnp.full((LANES,), h, jnp.int32), axis=0)
                    for h in range(heads)]
                for q in range(d // LANES):
                    cq = iotav + q * LANES
                    xv = plsc.load_gather(featv, [ssplat + cq])
                    plsc.addupdate_scatter(accf, [dsplat + cq],
                                           xv * exs[(q * LANES) // ch])

            @plsc.parallel_loop(0, j, unroll=2)
            def node_body(nd):
                rec = []
                for h in range(heads):
                    dh = plsc.load_gather(
                        accf,
                        [jnp.full((LANES,), d + h, jnp.int32) + nd * F])
                    rec.append(1.0 / (dh + 1e-16))
                for q in range(d // LANES):
                    acc = accf[pl.ds(nd * F + q * LANES, LANES)]
                    outv[pl.ds(nd * d + q * LANES, LANES)] = (
                        acc * rec[(q * LANES) // ch]
                        + biasv[pl.ds(q * LANES, LANES)])
                    accf[pl.ds(nd * F + q * LANES, LANES)] = zv
                accf[pl.ds(nd * F + d, LANES)] = zv

            pltpu.sync_copy(outv, out_hbm.at[pl.ds(gid * jd, jd)])
            return carry

        lax.fori_loop(0, gpw, body, jnp.int32(0))

    return sc_k


def kernel(x, edge_index, W, att_src, att_dst, bias):
    b, t, j, d = x.shape
    n = b * t * j
    ng = b * t
    e = edge_index.shape[1]
    heads = att_src.shape[1]
    F = 144

    x_flat = x.reshape(n, d)
    asrc_mat = jax.scipy.linalg.block_diag(
        *[att_src[0, h, :, None] for h in range(heads)])
    adst_mat = jax.scipy.linalg.block_diag(
        *[att_dst[0, h, :, None] for h in range(heads)])
    wcat = jnp.concatenate(
        [W, W @ asrc_mat, W @ adst_mat,
         jnp.zeros((d, F - d - 2 * heads), jnp.float32)], axis=1)
    feat = _tc_matmul(x_flat, wcat)

    ne = e + j
    EPAD = ((ne + LANES - 1) // LANES) * LANES
    src = jnp.concatenate([edge_index[0],
                           jnp.arange(j, dtype=jnp.int32)])
    dst = jnp.concatenate([edge_index[1],
                           jnp.arange(j, dtype=jnp.int32)])
    pad = jnp.zeros((EPAD - ne,), jnp.int32)
    src = jnp.concatenate([src, pad])
    dst = jnp.concatenate([dst, pad])
    eint = jnp.concatenate([src * F, dst * F, dst * d, dst])
    maskf = (jnp.arange(EPAD) < ne).astype(jnp.float32)

    gpw = ng // NW
    sc_k = _make_sc_kernel(n, j, d, heads, F, EPAD, ne, gpw)
    out_f = sc_k(feat.reshape(-1), eint, maskf, bias)
    return out_f.reshape(b, t, j, d)
